# baseline (speedup 1.0000x reference)
import functools

import jax
import jax.numpy as jnp
from jax.experimental import pallas as pl

D_MODEL = 768
N_BR = 12
N_SH = 12
H_TOT = 144
DH = 64
BLK = 16
TOPK = 8
SINK = 64
T = 1024
NB = T // BLK
SCALE = DH ** -0.5
NEG = -1e30
GH = 16

_INTERPRET = False


def _kv_base_kernel(x_ref, wk_ref, bk_ref, wv_ref, bv_ref, kb_ref, vb_ref,
                    cs_ref):
    pos = jax.lax.broadcasted_iota(jnp.int32, (T, DH // 2), 0).astype(jnp.float32)
    j2 = jax.lax.broadcasted_iota(jnp.int32, (T, DH // 2), 1).astype(jnp.float32)
    inv_freq = jnp.exp(j2 * (-2.0 / DH * jnp.log(jnp.float32(10000.0))))
    freqs = pos * inv_freq
    cs_ref[...] = jnp.concatenate((jnp.cos(freqs), jnp.sin(freqs)), axis=1)

    x = x_ref[...]
    kb = jax.lax.dot_general(
        x, wk_ref[...], (((1,), (1,)), ((), ())),
        preferred_element_type=jnp.float32) + bk_ref[...]
    vb = jax.lax.dot_general(
        x, wv_ref[...], (((1,), (1,)), ((), ())),
        preferred_element_type=jnp.float32) + bv_ref[...]
    for sh in range(N_SH):
        kb_ref[sh] = kb[:, sh * DH:(sh + 1) * DH]
        vb_ref[sh] = vb[:, sh * DH:(sh + 1) * DH]


def _qk_head_kernel(x_ref, wq_ref, kb_ref, wa_ref, idb_ref, cs_ref,
                    q_ref, k_ref, keep_ref):
    r = jax.lax.broadcasted_iota(jnp.int32, (DH, DH), 0)
    c = jax.lax.broadcasted_iota(jnp.int32, (DH, DH), 1)
    A = wa_ref[...]
    Askew = A - A.T

    cos = cs_ref[:, :DH // 2]
    sin = cs_ref[:, DH // 2:]

    rr = jax.lax.broadcasted_iota(jnp.int32, (NB, NB), 0)
    cc = jax.lax.broadcasted_iota(jnp.int32, (NB, NB), 1)

    def rope(p):
        p1 = p[:, :32]
        p2 = p[:, 32:]
        return jnp.concatenate(
            (p1 * cos - p2 * sin, p1 * sin + p2 * cos), axis=1)

    q_all = jax.lax.dot_general(
        x_ref[...], wq_ref[...], (((1,), (1,)), ((), ())),
        preferred_element_type=jnp.float32)

    t0 = pl.program_id(0) * GH
    for g in range(GH):
        Sg = Askew + jnp.where(r == c, idb_ref[0, g:g + 1, :], 0.0)
        qg = q_all[:, g * DH:(g + 1) * DH]
        q = rope(qg + jnp.dot(qg, Sg, preferred_element_type=jnp.float32))
        kb = kb_ref[(t0 + g) % N_SH]
        k = rope(kb + jnp.dot(kb, Sg, preferred_element_type=jnp.float32))
        q_ref[g] = q
        k_ref[g] = k

        qm = jnp.mean(q.reshape(NB, BLK, DH), axis=1)
        km = jnp.mean(k.reshape(NB, BLK, DH), axis=1)
        s = jax.lax.dot_general(qm, km, (((1,), (1,)), ((), ())),
                                preferred_element_type=jnp.float32)
        s = jnp.where(cc > rr, NEG, s)

        keep = jnp.zeros((NB, NB), jnp.float32)
        for _ in range(TOPK):
            hit = s >= jnp.max(s, axis=1, keepdims=True)
            keep = jnp.where(hit, 1.0, keep)
            s = jnp.where(hit, NEG, s)
        keep_ref[g] = keep


def _attn_kernel(q_ref, k_ref, v_ref, keep_ref, sink_ref, vn_ref, o_ref,
                 *, bq):
    qc = pl.program_id(1)
    qb = q_ref[0]
    s_h = sink_ref[0, 0, 0]
    nbq = bq // BLK

    LOG2E = 1.4426950408889634
    C2 = SCALE * LOG2E
    PENS = 2048.0

    kr = jax.lax.broadcasted_iota(jnp.int32, (nbq, NB), 0)
    kc_ = jax.lax.broadcasted_iota(jnp.int32, (nbq, NB), 1)
    keep2 = jnp.maximum(keep_ref[0],
                        jnp.where((kc_ < SINK // BLK) |
                                  (kc_ == qc * nbq + kr), 1.0, 0.0))

    ri = jax.lax.broadcasted_iota(jnp.int32, (bq, nbq), 0)
    rb = jax.lax.broadcasted_iota(jnp.int32, (bq, nbq), 1)
    R = jnp.where(ri // BLK == rb, 1.0, 0.0).astype(jnp.float32)
    qfeat = jnp.dot(R, PENS * (keep2 - 1.0),
                    preferred_element_type=jnp.float32)
    qb_aug = jnp.concatenate((qb * C2, qfeat), axis=1)

    dij = (jax.lax.broadcasted_iota(jnp.int32, (bq, bq), 0)
           - jax.lax.broadcasted_iota(jnp.int32, (bq, bq), 1))
    caus = dij >= 0
    loc = dij <= BLK
    jblk = jax.lax.broadcasted_iota(jnp.int32, (bq, NB), 0) // BLK
    bcol = jax.lax.broadcasted_iota(jnp.int32, (bq, NB), 1)

    def scores(kc):
        koff = kc * bq
        kchunk = k_ref[0, pl.ds(koff, bq), :]
        k1hot = jnp.where(jblk == bcol - kc * nbq, 1.0, 0.0)
        k_aug = jnp.concatenate((kchunk, k1hot), axis=1)
        return jax.lax.dot_general(qb_aug, k_aug, (((1,), (1,)), ((), ())),
                                   preferred_element_type=jnp.float32), koff

    def flash(s, koff, carry):
        l, acc = carry
        p = jnp.exp2(s)
        l = l + jnp.sum(p, axis=1, keepdims=True)
        vchunk = v_ref[0, pl.ds(koff, bq), :]
        acc = acc + jnp.dot(p, vchunk,
                            preferred_element_type=jnp.float32)
        return l, acc

    def body(kc, carry):
        s, koff = scores(kc)
        d = dij + (qc - kc) * bq
        is_keep = s > -0.5 * PENS
        s = jnp.where((d <= BLK) & (~is_keep), s + PENS, s)
        return flash(s, koff, carry)

    sh2 = s_h * LOG2E
    p_sink = jnp.exp2(jnp.zeros((bq, 1), jnp.float32) + sh2)
    l0 = p_sink
    acc0 = jnp.zeros((bq, DH), jnp.float32)
    carry = jax.lax.fori_loop(0, qc, body, (l0, acc0))

    s, koff = scores(qc)
    is_keep = s > -0.5 * PENS
    s = jnp.where(caus & (is_keep | loc),
                  jnp.where(is_keep, s, s + PENS), NEG)
    l, acc = flash(s, koff, carry)

    out = (acc + p_sink * vn_ref[0]) / l
    o_ref[0] = out


def _out_proj_kernel(ctx_ref, w_ref, b_ref, y_ref, *, bt):
    n = pl.program_id(1)
    ctx = jnp.concatenate([ctx_ref[sh] for sh in range(N_SH)], axis=1)
    acc = jnp.dot(ctx, w_ref[0], preferred_element_type=jnp.float32)
    acc = acc + b_ref[pl.ds(n, 1), :]

    @pl.when(n == 0)
    def _():
        y_ref[...] = acc

    @pl.when(n > 0)
    def _():
        y_ref[...] = y_ref[...] + acc


def kernel(x, W_Q_all, W_K, b_K, W_V, b_V, wedge_A, wedge_id_bias,
           sink_scalars, v_nulls, W_O_params, W_O_bias):
    xb = x.reshape(T, D_MODEL)

    pidx = jnp.concatenate([jnp.arange(0, DH, 2), jnp.arange(1, DH, 2)])
    wq_p = W_Q_all.reshape(H_TOT, DH, D_MODEL)[:, pidx, :].reshape(
        H_TOT * DH, D_MODEL)
    wk_p = W_K.reshape(N_SH, DH, D_MODEL)[:, pidx, :].reshape(
        D_MODEL, D_MODEL)
    bk_p = b_K.reshape(N_SH, DH)[:, pidx].reshape(1, D_MODEL)
    wa_p = wedge_A[pidx][:, pidx]
    idb_p = wedge_id_bias[:, pidx].reshape(H_TOT // GH, GH, DH)

    kb_vb = pl.pallas_call(
        _kv_base_kernel,
        grid=(1,),
        in_specs=[
            pl.BlockSpec((T, D_MODEL), lambda i: (0, 0)),
            pl.BlockSpec((D_MODEL, D_MODEL), lambda i: (0, 0)),
            pl.BlockSpec((1, D_MODEL), lambda i: (0, 0)),
            pl.BlockSpec((D_MODEL, D_MODEL), lambda i: (0, 0)),
            pl.BlockSpec((1, D_MODEL), lambda i: (0, 0)),
        ],
        out_specs=[
            pl.BlockSpec((N_SH, T, DH), lambda i: (0, 0, 0)),
            pl.BlockSpec((N_SH, T, DH), lambda i: (0, 0, 0)),
            pl.BlockSpec((T, DH), lambda i: (0, 0)),
        ],
        out_shape=[
            jax.ShapeDtypeStruct((N_SH, T, DH), jnp.float32),
            jax.ShapeDtypeStruct((N_SH, T, DH), jnp.float32),
            jax.ShapeDtypeStruct((T, DH), jnp.float32),
        ],
        interpret=_INTERPRET,
    )(xb, wk_p, bk_p, W_V, b_V.reshape(1, D_MODEL))
    k_base, v_base, cs_tab = kb_vb

    q, k, keep = pl.pallas_call(
        _qk_head_kernel,
        grid=(H_TOT // GH,),
        in_specs=[
            pl.BlockSpec((T, D_MODEL), lambda t: (0, 0)),
            pl.BlockSpec((GH * DH, D_MODEL), lambda t: (t, 0)),
            pl.BlockSpec((N_SH, T, DH), lambda t: (0, 0, 0)),
            pl.BlockSpec((DH, DH), lambda t: (0, 0)),
            pl.BlockSpec((1, GH, DH), lambda t: (t, 0, 0)),
            pl.BlockSpec((T, DH), lambda t: (0, 0)),
        ],
        out_specs=[
            pl.BlockSpec((GH, T, DH), lambda t: (t, 0, 0)),
            pl.BlockSpec((GH, T, DH), lambda t: (t, 0, 0)),
            pl.BlockSpec((GH, NB, NB), lambda t: (t, 0, 0)),
        ],
        out_shape=[
            jax.ShapeDtypeStruct((H_TOT, T, DH), jnp.float32),
            jax.ShapeDtypeStruct((H_TOT, T, DH), jnp.float32),
            jax.ShapeDtypeStruct((H_TOT, NB, NB), jnp.float32),
        ],
        interpret=_INTERPRET,
    )(xb, wq_p, k_base, wa_p, idb_p.reshape(H_TOT // GH, GH, DH), cs_tab)

    BQ = 256
    nqc = T // BQ
    sink3 = jnp.broadcast_to(sink_scalars.reshape(H_TOT, 1, 1),
                             (H_TOT, 1, DH))
    vn3 = v_nulls.reshape(H_TOT, 1, DH)
    ctx = pl.pallas_call(
        functools.partial(_attn_kernel, bq=BQ),
        grid=(H_TOT, nqc),
        in_specs=[
            pl.BlockSpec((1, BQ, DH), lambda h, qc: (h, qc, 0)),
            pl.BlockSpec((1, T, DH), lambda h, qc: (h, 0, 0)),
            pl.BlockSpec((1, T, DH), lambda h, qc: (h % N_SH, 0, 0)),
            pl.BlockSpec((1, BQ // BLK, NB), lambda h, qc: (h, qc, 0)),
            pl.BlockSpec((1, 1, DH), lambda h, qc: (h, 0, 0)),
            pl.BlockSpec((1, 1, DH), lambda h, qc: (h, 0, 0)),
        ],
        out_specs=pl.BlockSpec((1, BQ, DH), lambda h, qc: (h, qc, 0)),
        out_shape=jax.ShapeDtypeStruct((H_TOT, T, DH), jnp.float32),
        interpret=_INTERPRET,
    )(q, k, v_base, keep, sink3, vn3)

    BT = 256
    ntc = T // BT
    y = pl.pallas_call(
        functools.partial(_out_proj_kernel, bt=BT),
        grid=(ntc, N_BR),
        in_specs=[
            pl.BlockSpec((N_SH, BT, DH), lambda tc, n: (n, tc, 0)),
            pl.BlockSpec((1, D_MODEL, D_MODEL), lambda tc, n: (n, 0, 0)),
            pl.BlockSpec((N_BR, D_MODEL), lambda tc, n: (0, 0)),
        ],
        out_specs=pl.BlockSpec((BT, D_MODEL), lambda tc, n: (tc, 0)),
        out_shape=jax.ShapeDtypeStruct((T, D_MODEL), jnp.float32),
        interpret=_INTERPRET,
    )(ctx, W_O_params, W_O_bias)

    return y.reshape(1, T, D_MODEL)


# speedup vs baseline: 1.0159x; 1.0159x over previous
import functools

import jax
import jax.numpy as jnp
from jax.experimental import pallas as pl

D_MODEL = 768
N_BR = 12
N_SH = 12
H_TOT = 144
DH = 64
BLK = 16
TOPK = 8
SINK = 64
T = 1024
NB = T // BLK
SCALE = DH ** -0.5
NEG = -1e30
GH = 8

_INTERPRET = False


def _kv_base_kernel(x_ref, wk_ref, bk_ref, wv_ref, bv_ref, kb_ref, vb_ref,
                    cs_ref):
    pos = jax.lax.broadcasted_iota(jnp.int32, (T, DH // 2), 0).astype(jnp.float32)
    j2 = jax.lax.broadcasted_iota(jnp.int32, (T, DH // 2), 1).astype(jnp.float32)
    inv_freq = jnp.exp(j2 * (-2.0 / DH * jnp.log(jnp.float32(10000.0))))
    freqs = pos * inv_freq
    cs_ref[...] = jnp.concatenate((jnp.cos(freqs), jnp.sin(freqs)), axis=1)

    x = x_ref[...]
    kb = jax.lax.dot_general(
        x, wk_ref[...], (((1,), (1,)), ((), ())),
        preferred_element_type=jnp.float32) + bk_ref[...]
    vb = jax.lax.dot_general(
        x, wv_ref[...], (((1,), (1,)), ((), ())),
        preferred_element_type=jnp.float32) + bv_ref[...]
    for sh in range(N_SH):
        kb_ref[sh] = kb[:, sh * DH:(sh + 1) * DH]
        vb_ref[sh] = vb[:, sh * DH:(sh + 1) * DH]


def _qk_head_kernel(x_ref, wq_ref, kb_ref, wa_ref, idb_ref, cs_ref,
                    q_ref, k_ref, keep_ref):
    r = jax.lax.broadcasted_iota(jnp.int32, (DH, DH), 0)
    c = jax.lax.broadcasted_iota(jnp.int32, (DH, DH), 1)
    A = wa_ref[...]
    Askew = A - A.T

    cos = cs_ref[:, :DH // 2]
    sin = cs_ref[:, DH // 2:]

    rr = jax.lax.broadcasted_iota(jnp.int32, (NB, NB), 0)
    cc = jax.lax.broadcasted_iota(jnp.int32, (NB, NB), 1)

    def rope(p):
        p1 = p[:, :32]
        p2 = p[:, 32:]
        return jnp.concatenate(
            (p1 * cos - p2 * sin, p1 * sin + p2 * cos), axis=1)

    q_all = jax.lax.dot_general(
        x_ref[...], wq_ref[...], (((1,), (1,)), ((), ())),
        preferred_element_type=jnp.float32)

    t0 = pl.program_id(0) * GH
    for g in range(GH):
        Sg = Askew + jnp.where(r == c, idb_ref[0, g:g + 1, :], 0.0)
        qg = q_all[:, g * DH:(g + 1) * DH]
        q = rope(qg + jnp.dot(qg, Sg, preferred_element_type=jnp.float32))
        kb = kb_ref[(t0 + g) % N_SH]
        k = rope(kb + jnp.dot(kb, Sg, preferred_element_type=jnp.float32))
        q_ref[g] = q
        k_ref[g] = k

        qm = jnp.mean(q.reshape(NB, BLK, DH), axis=1)
        km = jnp.mean(k.reshape(NB, BLK, DH), axis=1)
        s = jax.lax.dot_general(qm, km, (((1,), (1,)), ((), ())),
                                preferred_element_type=jnp.float32)
        s = jnp.where(cc > rr, NEG, s)

        keep = jnp.zeros((NB, NB), jnp.float32)
        for _ in range(TOPK):
            hit = s >= jnp.max(s, axis=1, keepdims=True)
            keep = jnp.where(hit, 1.0, keep)
            s = jnp.where(hit, NEG, s)
        keep_ref[g] = keep


def _attn_kernel(q_ref, k_ref, v_ref, keep_ref, sink_ref, vn_ref, o_ref,
                 *, bq):
    qc = pl.program_id(1)
    qb = q_ref[0]
    s_h = sink_ref[0, 0, 0]
    nbq = bq // BLK

    LOG2E = 1.4426950408889634
    C2 = SCALE * LOG2E
    PENS = 2048.0

    kr = jax.lax.broadcasted_iota(jnp.int32, (nbq, NB), 0)
    kc_ = jax.lax.broadcasted_iota(jnp.int32, (nbq, NB), 1)
    keep2 = jnp.maximum(keep_ref[0],
                        jnp.where((kc_ < SINK // BLK) |
                                  (kc_ == qc * nbq + kr), 1.0, 0.0))

    ri = jax.lax.broadcasted_iota(jnp.int32, (bq, nbq), 0)
    rb = jax.lax.broadcasted_iota(jnp.int32, (bq, nbq), 1)
    R = jnp.where(ri // BLK == rb, 1.0, 0.0).astype(jnp.float32)
    qfeat = jnp.dot(R, PENS * (keep2 - 1.0),
                    preferred_element_type=jnp.float32)
    qb_aug = jnp.concatenate((qb * C2, qfeat), axis=1)

    dij = (jax.lax.broadcasted_iota(jnp.int32, (bq, bq), 0)
           - jax.lax.broadcasted_iota(jnp.int32, (bq, bq), 1))
    caus = dij >= 0
    loc = dij <= BLK
    jblk = jax.lax.broadcasted_iota(jnp.int32, (bq, NB), 0) // BLK
    bcol = jax.lax.broadcasted_iota(jnp.int32, (bq, NB), 1)

    def scores(kc):
        koff = kc * bq
        kchunk = k_ref[0, pl.ds(koff, bq), :]
        k1hot = jnp.where(jblk == bcol - kc * nbq, 1.0, 0.0)
        k_aug = jnp.concatenate((kchunk, k1hot), axis=1)
        return jax.lax.dot_general(qb_aug, k_aug, (((1,), (1,)), ((), ())),
                                   preferred_element_type=jnp.float32), koff

    def flash(s, koff, carry):
        l, acc = carry
        p = jnp.exp2(s)
        l = l + jnp.sum(p, axis=1, keepdims=True)
        vchunk = v_ref[0, pl.ds(koff, bq), :]
        acc = acc + jnp.dot(p, vchunk,
                            preferred_element_type=jnp.float32)
        return l, acc

    def body(kc, carry):
        s, koff = scores(kc)
        d = dij + (qc - kc) * bq
        is_keep = s > -0.5 * PENS
        s = jnp.where((d <= BLK) & (~is_keep), s + PENS, s)
        return flash(s, koff, carry)

    sh2 = s_h * LOG2E
    p_sink = jnp.exp2(jnp.zeros((bq, 1), jnp.float32) + sh2)
    l0 = p_sink
    acc0 = jnp.zeros((bq, DH), jnp.float32)
    carry = jax.lax.fori_loop(0, qc, body, (l0, acc0))

    s, koff = scores(qc)
    is_keep = s > -0.5 * PENS
    s = jnp.where(caus & (is_keep | loc),
                  jnp.where(is_keep, s, s + PENS), NEG)
    l, acc = flash(s, koff, carry)

    out = (acc + p_sink * vn_ref[0]) / l
    o_ref[0] = out


def _out_proj_kernel(ctx_ref, w_ref, b_ref, y_ref, *, bt):
    n = pl.program_id(1)
    ctx = jnp.concatenate([ctx_ref[sh] for sh in range(N_SH)], axis=1)
    acc = jnp.dot(ctx, w_ref[0], preferred_element_type=jnp.float32)
    acc = acc + b_ref[pl.ds(n, 1), :]

    @pl.when(n == 0)
    def _():
        y_ref[...] = acc

    @pl.when(n > 0)
    def _():
        y_ref[...] = y_ref[...] + acc


def kernel(x, W_Q_all, W_K, b_K, W_V, b_V, wedge_A, wedge_id_bias,
           sink_scalars, v_nulls, W_O_params, W_O_bias):
    xb = x.reshape(T, D_MODEL)

    pidx = jnp.concatenate([jnp.arange(0, DH, 2), jnp.arange(1, DH, 2)])
    wq_p = W_Q_all.reshape(H_TOT, DH, D_MODEL)[:, pidx, :].reshape(
        H_TOT * DH, D_MODEL)
    wk_p = W_K.reshape(N_SH, DH, D_MODEL)[:, pidx, :].reshape(
        D_MODEL, D_MODEL)
    bk_p = b_K.reshape(N_SH, DH)[:, pidx].reshape(1, D_MODEL)
    wa_p = wedge_A[pidx][:, pidx]
    idb_p = wedge_id_bias[:, pidx].reshape(H_TOT // GH, GH, DH)

    kb_vb = pl.pallas_call(
        _kv_base_kernel,
        grid=(1,),
        in_specs=[
            pl.BlockSpec((T, D_MODEL), lambda i: (0, 0)),
            pl.BlockSpec((D_MODEL, D_MODEL), lambda i: (0, 0)),
            pl.BlockSpec((1, D_MODEL), lambda i: (0, 0)),
            pl.BlockSpec((D_MODEL, D_MODEL), lambda i: (0, 0)),
            pl.BlockSpec((1, D_MODEL), lambda i: (0, 0)),
        ],
        out_specs=[
            pl.BlockSpec((N_SH, T, DH), lambda i: (0, 0, 0)),
            pl.BlockSpec((N_SH, T, DH), lambda i: (0, 0, 0)),
            pl.BlockSpec((T, DH), lambda i: (0, 0)),
        ],
        out_shape=[
            jax.ShapeDtypeStruct((N_SH, T, DH), jnp.float32),
            jax.ShapeDtypeStruct((N_SH, T, DH), jnp.float32),
            jax.ShapeDtypeStruct((T, DH), jnp.float32),
        ],
        interpret=_INTERPRET,
    )(xb, wk_p, bk_p, W_V, b_V.reshape(1, D_MODEL))
    k_base, v_base, cs_tab = kb_vb

    q, k, keep = pl.pallas_call(
        _qk_head_kernel,
        grid=(H_TOT // GH,),
        in_specs=[
            pl.BlockSpec((T, D_MODEL), lambda t: (0, 0)),
            pl.BlockSpec((GH * DH, D_MODEL), lambda t: (t, 0)),
            pl.BlockSpec((N_SH, T, DH), lambda t: (0, 0, 0)),
            pl.BlockSpec((DH, DH), lambda t: (0, 0)),
            pl.BlockSpec((1, GH, DH), lambda t: (t, 0, 0)),
            pl.BlockSpec((T, DH), lambda t: (0, 0)),
        ],
        out_specs=[
            pl.BlockSpec((GH, T, DH), lambda t: (t, 0, 0)),
            pl.BlockSpec((GH, T, DH), lambda t: (t, 0, 0)),
            pl.BlockSpec((GH, NB, NB), lambda t: (t, 0, 0)),
        ],
        out_shape=[
            jax.ShapeDtypeStruct((H_TOT, T, DH), jnp.float32),
            jax.ShapeDtypeStruct((H_TOT, T, DH), jnp.float32),
            jax.ShapeDtypeStruct((H_TOT, NB, NB), jnp.float32),
        ],
        interpret=_INTERPRET,
    )(xb, wq_p, k_base, wa_p, idb_p.reshape(H_TOT // GH, GH, DH), cs_tab)

    BQ = 256
    nqc = T // BQ
    sink3 = jnp.broadcast_to(sink_scalars.reshape(H_TOT, 1, 1),
                             (H_TOT, 1, DH))
    vn3 = v_nulls.reshape(H_TOT, 1, DH)
    ctx = pl.pallas_call(
        functools.partial(_attn_kernel, bq=BQ),
        grid=(H_TOT, nqc),
        in_specs=[
            pl.BlockSpec((1, BQ, DH), lambda h, qc: (h, qc, 0)),
            pl.BlockSpec((1, T, DH), lambda h, qc: (h, 0, 0)),
            pl.BlockSpec((1, T, DH), lambda h, qc: (h % N_SH, 0, 0)),
            pl.BlockSpec((1, BQ // BLK, NB), lambda h, qc: (h, qc, 0)),
            pl.BlockSpec((1, 1, DH), lambda h, qc: (h, 0, 0)),
            pl.BlockSpec((1, 1, DH), lambda h, qc: (h, 0, 0)),
        ],
        out_specs=pl.BlockSpec((1, BQ, DH), lambda h, qc: (h, qc, 0)),
        out_shape=jax.ShapeDtypeStruct((H_TOT, T, DH), jnp.float32),
        interpret=_INTERPRET,
    )(q, k, v_base, keep, sink3, vn3)

    BT = 256
    ntc = T // BT
    y = pl.pallas_call(
        functools.partial(_out_proj_kernel, bt=BT),
        grid=(ntc, N_BR),
        in_specs=[
            pl.BlockSpec((N_SH, BT, DH), lambda tc, n: (n, tc, 0)),
            pl.BlockSpec((1, D_MODEL, D_MODEL), lambda tc, n: (n, 0, 0)),
            pl.BlockSpec((N_BR, D_MODEL), lambda tc, n: (0, 0)),
        ],
        out_specs=pl.BlockSpec((BT, D_MODEL), lambda tc, n: (tc, 0)),
        out_shape=jax.ShapeDtypeStruct((T, D_MODEL), jnp.float32),
        interpret=_INTERPRET,
    )(ctx, W_O_params, W_O_bias)

    return y.reshape(1, T, D_MODEL)
